# Initial kernel scaffold; baseline (speedup 1.0000x reference)
import functools

import jax
import jax.numpy as jnp
from jax import lax
from jax.experimental import pallas as pl
from jax.experimental.pallas import tpu as pltpu
from jax.experimental.pallas import tpu_sc as plsc

N_NODES = 10000
N_EDGES = 320000
D_IN = 128
D_HID = 128
D_OUT = 64

NC = 2
NS = 16
NW = NC * NS
B = 128
T = -(-N_EDGES // (NW * B))
E_PAD = NW * T * B
N_PAD = 10240
RPS = N_PAD // NS
ZCH = RPS // B


def _sc_agg_body(D, with_deg, *refs):
    if with_deg:
        (z_hbm, src_hbm, dst_hbm, ones_hbm, zrows_hbm, z16_hbm,
         acc_out, deg_out, srcb, dstb, rows, onesv, v16, acc_sh, deg_sh,
         sem) = refs
    else:
        (z_hbm, src_hbm, dst_hbm, zrows_hbm,
         acc_out, srcb, dstb, rows, acc_sh, sem) = refs
    c = lax.axis_index("c")
    s = lax.axis_index("s")
    w = s * NC + c

    pltpu.sync_copy(zrows_hbm, rows)
    if with_deg:
        pltpu.sync_copy(z16_hbm, v16)

    def zbody(i, _):
        r0 = s * RPS + i * B
        pltpu.sync_copy(rows, acc_sh.at[pl.ds(r0, B)])
        if with_deg:
            pltpu.sync_copy(v16, deg_sh.at[pl.ds(r0, B)])
        return _

    lax.fori_loop(0, ZCH, zbody, None)
    if with_deg:
        pltpu.sync_copy(ones_hbm, onesv)

    pltpu.sync_copy(src_hbm.at[w], srcb)
    pltpu.sync_copy(dst_hbm.at[w], dstb)
    plsc.subcore_barrier()

    def ebody(j, _):
        pltpu.async_copy(z_hbm.at[srcb.at[j]], rows, sem).wait()
        pltpu.sync_copy(rows, acc_sh.at[dstb.at[j]], add=True)
        if with_deg:
            pltpu.sync_copy(onesv, deg_sh.at[dstb.at[j]], add=True)
        return _

    lax.fori_loop(0, T, ebody, None)
    plsc.subcore_barrier()

    def obody(i, _):
        r0 = s * RPS + i * B
        pltpu.sync_copy(acc_sh.at[pl.ds(r0, B)], rows)
        pltpu.sync_copy(rows, acc_out.at[c].at[pl.ds(r0, B)])
        if with_deg:
            pltpu.sync_copy(deg_sh.at[pl.ds(r0, B)], v16)
            pltpu.sync_copy(v16, deg_out.at[c].at[pl.ds(r0, B)])
        return _

    lax.fori_loop(0, ZCH, obody, None)


def _make_sc_agg(D, with_deg):
    mesh = plsc.VectorSubcoreMesh(core_axis_name="c", subcore_axis_name="s")
    out_type = [jax.ShapeDtypeStruct((NC, N_PAD, D), jnp.float32)]
    scratch = [
        pltpu.VMEM((T, B), jnp.int32),
        pltpu.VMEM((T, B), jnp.int32),
        pltpu.VMEM((B, D), jnp.float32),
    ]
    if with_deg:
        out_type.append(jax.ShapeDtypeStruct((NC, N_PAD, 16), jnp.float32))
        scratch += [
            pltpu.VMEM((B, 16), jnp.float32),
            pltpu.VMEM((B, 16), jnp.float32),
        ]
    scratch.append(pltpu.VMEM_SHARED((N_PAD, D), jnp.float32))
    if with_deg:
        scratch.append(pltpu.VMEM_SHARED((N_PAD, 16), jnp.float32))
    scratch.append(pltpu.SemaphoreType.DMA)
    return pl.kernel(
        functools.partial(_sc_agg_body, D, with_deg),
        out_type=tuple(out_type),
        mesh=mesh,
        scratch_types=tuple(scratch),
    )


def _tc_mid_body(x_ref, acc_ref, deg_ref, ws1, wn1, b1, ws2, wn2, b2,
                 z2_ref, s2_ref):
    x = x_ref[...]
    agg = acc_ref[0] + acc_ref[1]
    deg = jnp.maximum(deg_ref[0, :, 0:1] + deg_ref[1, :, 0:1], 1.0)
    h_n = agg / deg
    h1 = x @ ws1[...] + h_n @ wn1[...] + b1[...]
    h1 = jnp.maximum(h1, 0.0)
    z2_ref[...] = h1 @ wn2[...]
    s2_ref[...] = h1 @ ws2[...] + b2[...]


def _tc_out_body(acc_ref, deg_ref, s2_ref, out_ref):
    agg = acc_ref[0] + acc_ref[1]
    deg = jnp.maximum(deg_ref[0, :, 0:1] + deg_ref[1, :, 0:1], 1.0)
    out_ref[...] = s2_ref[...] + agg / deg


_R = 1000


def kernel(node_features, edge_index, W_self1, W_neigh1, b1,
           W_self2, W_neigh2, b2):
    src = edge_index[0].astype(jnp.int32)
    dst = edge_index[1].astype(jnp.int32)
    pad = E_PAD - N_EDGES
    src3 = jnp.concatenate(
        [src, jnp.zeros((pad,), jnp.int32)]).reshape(NW, T, B)
    dst3 = jnp.concatenate(
        [dst, jnp.full((pad,), N_NODES, jnp.int32)]).reshape(NW, T, B)
    ones16 = jnp.ones((B, 16), jnp.float32)
    zrows128 = jnp.zeros((B, D_IN), jnp.float32)
    zrows64 = jnp.zeros((B, D_OUT), jnp.float32)
    z16 = jnp.zeros((B, 16), jnp.float32)

    acc1, deg = _make_sc_agg(D_IN, True)(
        node_features, src3, dst3, ones16, zrows128, z16)

    grid = N_NODES // _R
    full = lambda i: (0, 0)
    z2, s2 = pl.pallas_call(
        _tc_mid_body,
        grid=(grid,),
        in_specs=[
            pl.BlockSpec((_R, D_IN), lambda i: (i, 0)),
            pl.BlockSpec((NC, _R, D_IN), lambda i: (0, i, 0)),
            pl.BlockSpec((NC, _R, 16), lambda i: (0, i, 0)),
            pl.BlockSpec((D_IN, D_HID), full),
            pl.BlockSpec((D_IN, D_HID), full),
            pl.BlockSpec((1, D_HID), full),
            pl.BlockSpec((D_HID, D_OUT), full),
            pl.BlockSpec((D_HID, D_OUT), full),
            pl.BlockSpec((1, D_OUT), full),
        ],
        out_specs=[
            pl.BlockSpec((_R, D_OUT), lambda i: (i, 0)),
            pl.BlockSpec((_R, D_OUT), lambda i: (i, 0)),
        ],
        out_shape=[
            jax.ShapeDtypeStruct((N_NODES, D_OUT), jnp.float32),
            jax.ShapeDtypeStruct((N_NODES, D_OUT), jnp.float32),
        ],
    )(node_features, acc1, deg, W_self1, W_neigh1, b1.reshape(1, D_HID),
      W_self2, W_neigh2, b2.reshape(1, D_OUT))

    (acc2,) = _make_sc_agg(D_OUT, False)(z2, src3, dst3, zrows64)

    out = pl.pallas_call(
        _tc_out_body,
        grid=(grid,),
        in_specs=[
            pl.BlockSpec((NC, _R, D_OUT), lambda i: (0, i, 0)),
            pl.BlockSpec((NC, _R, 16), lambda i: (0, i, 0)),
            pl.BlockSpec((_R, D_OUT), lambda i: (i, 0)),
        ],
        out_specs=pl.BlockSpec((_R, D_OUT), lambda i: (i, 0)),
        out_shape=jax.ShapeDtypeStruct((N_NODES, D_OUT), jnp.float32),
    )(acc2, deg, s2)
    return out


# speedup vs baseline: 5.4844x; 5.4844x over previous
import functools

import jax
import jax.numpy as jnp
from jax import lax
from jax.experimental import pallas as pl
from jax.experimental.pallas import tpu as pltpu
from jax.experimental.pallas import tpu_sc as plsc

N_NODES = 10000
N_EDGES = 320000
D_IN = 128
D_HID = 128
D_OUT = 64

NC = 2
NS = 16
NW = NC * NS
B = 128
T1 = 2 * -(-N_EDGES // (NS * B * 2))
T2 = T1 // 2
E_PAD = NS * T1 * B
N_PAD = 10240
RPS = N_PAD // NS
ZCH = RPS // B
HA = 80


def _sc_agg_body(H, T, col_split, *refs):
    (z_hbm, src_hbm, dst_hbm, zrows_hbm,
     acc_out, srcb, dstb, rows, acc_sh, sem) = refs
    c = lax.axis_index("c")
    s = lax.axis_index("s")

    pltpu.sync_copy(zrows_hbm, rows)

    def zbody(i, _):
        pltpu.sync_copy(rows, acc_sh.at[pl.ds(s * RPS + i * B, B)])
        return _

    lax.fori_loop(0, ZCH, zbody, None)

    chunk = s if col_split else s * NC + c
    pltpu.sync_copy(src_hbm.at[chunk], srcb)
    pltpu.sync_copy(dst_hbm.at[chunk], dstb)
    plsc.subcore_barrier()

    gather_src = z_hbm.at[c] if col_split else z_hbm

    def ebody(j, _):
        pltpu.async_copy(gather_src.at[srcb.at[j]], rows, sem).wait()
        pltpu.sync_copy(rows, acc_sh.at[dstb.at[j]], add=True)
        return _

    lax.fori_loop(0, T, ebody, None)
    plsc.subcore_barrier()

    def obody(i, _):
        r0 = s * RPS + i * B
        pltpu.sync_copy(acc_sh.at[pl.ds(r0, B)], rows)
        pltpu.sync_copy(rows, acc_out.at[c].at[pl.ds(r0, B)])
        return _

    lax.fori_loop(0, ZCH, obody, None)


def _make_sc_agg(H, T, col_split):
    mesh = plsc.VectorSubcoreMesh(core_axis_name="c", subcore_axis_name="s")
    return pl.kernel(
        functools.partial(_sc_agg_body, H, T, col_split),
        out_type=(jax.ShapeDtypeStruct((NC, N_PAD, H), jnp.float32),),
        mesh=mesh,
        scratch_types=(
            pltpu.VMEM((T, B), jnp.int32),
            pltpu.VMEM((T, B), jnp.int32),
            pltpu.VMEM((B, H), jnp.float32),
            pltpu.VMEM_SHARED((N_PAD, H), jnp.float32),
            pltpu.SemaphoreType.DMA,
        ),
        compiler_params=pltpu.CompilerParams(use_tc_tiling_on_sc=False),
    )


def _tc_mid_body(x_ref, acc_ref, ws1, wn1, b1, ws2, wn2, b2,
                 z2_ref, s2_ref):
    x = x_ref[...]
    agg = jnp.concatenate([acc_ref[0, :, :64], acc_ref[1, :, :64]], axis=-1)
    deg = jnp.maximum(acc_ref[0, :, 64:65], 1.0)
    h_n = agg / deg
    dot = functools.partial(jnp.dot, precision=lax.Precision.HIGHEST,
                            preferred_element_type=jnp.float32)
    h1 = dot(x, ws1[...]) + dot(h_n, wn1[...]) + b1[...]
    h1 = jnp.maximum(h1, 0.0)
    z2_ref[...] = dot(h1, wn2[...])
    s2_ref[...] = dot(h1, ws2[...]) + b2[...]


def _tc_out_body(acc2_ref, acc1_ref, s2_ref, out_ref):
    agg = acc2_ref[0] + acc2_ref[1]
    deg = jnp.maximum(acc1_ref[0, :, 64:65], 1.0)
    out_ref[...] = s2_ref[...] + agg / deg


_R = 1000


def kernel(node_features, edge_index, W_self1, W_neigh1, b1,
           W_self2, W_neigh2, b2):
    src = edge_index[0].astype(jnp.int32)
    dst = edge_index[1].astype(jnp.int32)
    pad = E_PAD - N_EDGES
    src_p = jnp.concatenate([src, jnp.zeros((pad,), jnp.int32)])
    dst_p = jnp.concatenate([dst, jnp.full((pad,), N_NODES, jnp.int32)])
    src_a, dst_a = src_p.reshape(NS, T1, B), dst_p.reshape(NS, T1, B)
    src_b, dst_b = src_p.reshape(NW, T2, B), dst_p.reshape(NW, T2, B)

    one = jnp.ones((N_NODES, 1), jnp.float32)
    zpad = jnp.zeros((N_NODES, HA - 65), jnp.float32)
    x_aug = jnp.stack([
        jnp.concatenate([node_features[:, :64], one, zpad], axis=1),
        jnp.concatenate([node_features[:, 64:], one, zpad], axis=1),
    ])

    (acc1,) = _make_sc_agg(HA, T1, True)(
        x_aug, src_a, dst_a, jnp.zeros((B, HA), jnp.float32))

    grid = N_NODES // _R
    full = lambda i: (0, 0)
    z2, s2 = pl.pallas_call(
        _tc_mid_body,
        grid=(grid,),
        in_specs=[
            pl.BlockSpec((_R, D_IN), lambda i: (i, 0)),
            pl.BlockSpec((NC, _R, HA), lambda i: (0, i, 0)),
            pl.BlockSpec((D_IN, D_HID), full),
            pl.BlockSpec((D_IN, D_HID), full),
            pl.BlockSpec((1, D_HID), full),
            pl.BlockSpec((D_HID, D_OUT), full),
            pl.BlockSpec((D_HID, D_OUT), full),
            pl.BlockSpec((1, D_OUT), full),
        ],
        out_specs=[
            pl.BlockSpec((_R, D_OUT), lambda i: (i, 0)),
            pl.BlockSpec((_R, D_OUT), lambda i: (i, 0)),
        ],
        out_shape=[
            jax.ShapeDtypeStruct((N_NODES, D_OUT), jnp.float32),
            jax.ShapeDtypeStruct((N_NODES, D_OUT), jnp.float32),
        ],
    )(node_features, acc1, W_self1, W_neigh1, b1.reshape(1, D_HID),
      W_self2, W_neigh2, b2.reshape(1, D_OUT))

    (acc2,) = _make_sc_agg(D_OUT, T2, False)(
        z2, src_b, dst_b, jnp.zeros((B, D_OUT), jnp.float32))

    out = pl.pallas_call(
        _tc_out_body,
        grid=(grid,),
        in_specs=[
            pl.BlockSpec((NC, _R, D_OUT), lambda i: (0, i, 0)),
            pl.BlockSpec((NC, _R, HA), lambda i: (0, i, 0)),
            pl.BlockSpec((_R, D_OUT), lambda i: (i, 0)),
        ],
        out_specs=pl.BlockSpec((_R, D_OUT), lambda i: (i, 0)),
        out_shape=jax.ShapeDtypeStruct((N_NODES, D_OUT), jnp.float32),
    )(acc2, acc1, s2)
    return out
